# baseline (speedup 1.0000x reference)
import functools

import jax
import jax.numpy as jnp
from jax import lax
from jax.experimental import pallas as pl
from jax.experimental.pallas import tpu as pltpu
from jax.experimental.pallas import tpu_sc as plsc

N = 100000
H = 128
G = 512
R = 8192
NBLK = 13
NPAD = NBLK * R
NC, NS = 2, 16
NW = NC * NS
CH = NPAD // NW
W = 64
NV = CH // 16
NEG = -1e30


def _k1_body(x_ref, wg1_ref, bg1_ref, wg2_ref, bg2_ref, g_ref, gmax_ref, mscr):
    i = pl.program_id(0)
    x = x_ref[...]
    h = jnp.dot(x, wg1_ref[...], preferred_element_type=jnp.float32) + bg1_ref[...]
    h = h * jax.nn.sigmoid(h)
    g = jnp.dot(h, wg2_ref[...], preferred_element_type=jnp.float32) + bg2_ref[...]
    g_ref[...] = g

    @pl.when(i == NBLK - 1)
    def _():
        rows = i * R + lax.broadcasted_iota(jnp.int32, (R, 1), 0)
        g_ref[...] = jnp.where(rows < N, g, NEG)

    @pl.when(i == 0)
    def _():
        mscr[...] = jnp.full((8, 128), NEG, jnp.float32)

    mscr[...] = jnp.maximum(mscr[...], jnp.max(g_ref[...]))

    @pl.when(i == NBLK - 1)
    def _():
        gmax_ref[...] = mscr[...]


def _k3_body(x_ref, e_ref, b_ref, base_ref, wide_ref, wv_ref, bv_ref,
             parts_ref, out_ref):
    i = pl.program_id(0)

    @pl.when(i == 0)
    def _():
        out_ref[...] = jnp.zeros((G, H), jnp.float32)

    x = x_ref[...]
    v = jnp.dot(x, wv_ref[...], preferred_element_type=jnp.float32) + bv_ref[...]
    v = v * jax.nn.sigmoid(v)
    rows = i * R + lax.broadcasted_iota(jnp.int32, (R, 1), 0)
    w = jnp.where(rows < N, v * e_ref[...], 0.0)
    b = b_ref[...]
    base = base_ref[i]
    iota_w = lax.broadcasted_iota(jnp.int32, (R, W), 1)

    def window_update(wbase):
        oh = (b == wbase + iota_w).astype(jnp.float32)
        out_ref[pl.ds(wbase, W), :] += lax.dot_general(
            oh, w, (((0,), (0,)), ((), ())),
            preferred_element_type=jnp.float32)

    @pl.when(wide_ref[i] == 0)
    def _():
        window_update(base)

    @pl.when(wide_ref[i] != 0)
    def _():
        for k in range(G // W):
            window_update(k * W)

    @pl.when(i == NBLK - 1)
    def _():
        den = jnp.sum(parts_ref[...], axis=0)
        out_ref[...] = out_ref[...] / (den[:, None] + 1e-16)


def _sc_softmax_body(g_hbm, b_hbm, gmax_hbm, e_hbm, parts_hbm,
                     g_v, b_v, e_v, den_v, gmax_v, sem0, sem1, sem2):
    wid = lax.axis_index("s") * NC + lax.axis_index("c")
    base = wid * CH
    c0 = pltpu.async_copy(g_hbm.at[pl.ds(base, CH)], g_v, sem0)
    c1 = pltpu.async_copy(b_hbm.at[pl.ds(base, CH)], b_v, sem1)
    c2 = pltpu.async_copy(gmax_hbm, gmax_v, sem2)
    z = jnp.zeros((16,), jnp.float32)
    for k in range(G // 16):
        den_v[pl.ds(k * 16, 16)] = z
    c0.wait()
    c1.wait()
    c2.wait()
    gm = gmax_v[...]

    def body(j, carry):
        sl = pl.ds(j * 16, 16)
        ev = jnp.exp(g_v[sl] - gm)
        e_v[sl] = ev
        plsc.addupdate_scatter(den_v, [b_v[sl]], ev)
        return carry

    lax.fori_loop(0, NV, body, 0)
    pltpu.sync_copy(e_v, e_hbm.at[pl.ds(base, CH)])
    pltpu.sync_copy(den_v, parts_hbm.at[wid])


@functools.cache
def _sc_softmax():
    return functools.partial(
        pl.kernel,
        mesh=plsc.VectorSubcoreMesh(core_axis_name="c", subcore_axis_name="s"),
        out_type=[
            jax.ShapeDtypeStruct((NPAD,), jnp.float32),
            jax.ShapeDtypeStruct((NW, G), jnp.float32),
        ],
        scratch_types=[
            pltpu.VMEM((CH,), jnp.float32),
            pltpu.VMEM((CH,), jnp.int32),
            pltpu.VMEM((CH,), jnp.float32),
            pltpu.VMEM((G,), jnp.float32),
            pltpu.VMEM((16,), jnp.float32),
            pltpu.SemaphoreType.DMA,
            pltpu.SemaphoreType.DMA,
            pltpu.SemaphoreType.DMA,
        ],
        compiler_params=pltpu.CompilerParams(needs_layout_passes=False),
    )(_sc_softmax_body)


def _gate_pass(x, Wg1, bg1, Wg2, bg2):
    return pl.pallas_call(
        _k1_body,
        grid=(NBLK,),
        in_specs=[
            pl.BlockSpec((R, H), lambda i: (i, 0)),
            pl.BlockSpec((H, H // 2), lambda i: (0, 0)),
            pl.BlockSpec((1, H // 2), lambda i: (0, 0)),
            pl.BlockSpec((H // 2, 1), lambda i: (0, 0)),
            pl.BlockSpec((1, 1), lambda i: (0, 0)),
        ],
        out_specs=[
            pl.BlockSpec((R, 1), lambda i: (i, 0)),
            pl.BlockSpec((8, 128), lambda i: (0, 0)),
        ],
        out_shape=[
            jax.ShapeDtypeStruct((NPAD, 1), jnp.float32),
            jax.ShapeDtypeStruct((8, 128), jnp.float32),
        ],
        scratch_shapes=[pltpu.VMEM((8, 128), jnp.float32)],
        compiler_params=pltpu.CompilerParams(
            dimension_semantics=("arbitrary",)),
    )(x, Wg1.T, bg1.reshape(1, H // 2), Wg2.T, bg2.reshape(1, 1))


def _pool_pass(x, e, batch2d, base, wide, Wv, bv, parts):
    return pl.pallas_call(
        _k3_body,
        grid=(NBLK,),
        in_specs=[
            pl.BlockSpec((R, H), lambda i: (i, 0)),
            pl.BlockSpec((R, 1), lambda i: (i, 0)),
            pl.BlockSpec((R, 1), lambda i: (i, 0)),
            pl.BlockSpec((NBLK,), lambda i: (0,), memory_space=pltpu.SMEM),
            pl.BlockSpec((NBLK,), lambda i: (0,), memory_space=pltpu.SMEM),
            pl.BlockSpec((H, H), lambda i: (0, 0)),
            pl.BlockSpec((1, H), lambda i: (0, 0)),
            pl.BlockSpec((NW, G), lambda i: (0, 0)),
        ],
        out_specs=pl.BlockSpec((G, H), lambda i: (0, 0)),
        out_shape=jax.ShapeDtypeStruct((G, H), jnp.float32),
        compiler_params=pltpu.CompilerParams(
            dimension_semantics=("arbitrary",)),
    )(x, e, batch2d, base, wide, Wv.T, bv.reshape(1, H), parts)


def kernel(x, batch, Wg1, bg1, Wg2, bg2, Wv, bv):
    batch_i = batch.astype(jnp.int32)
    batch_p = jnp.pad(batch_i, (0, NPAD - N), mode="edge")
    b2 = batch_p.reshape(NBLK, R)
    lo, hi = b2[:, 0], b2[:, -1]
    base = jnp.minimum(lo & -8, G - W)
    wide = (hi - base >= W).astype(jnp.int32)
    g, gmax = _gate_pass(x, Wg1, bg1, Wg2, bg2)
    e, parts = _sc_softmax()(g.reshape(NPAD), batch_p, gmax.reshape(-1)[:16])
    return _pool_pass(x, e.reshape(NPAD, 1), batch_p.reshape(NPAD, 1),
                      base, wide, Wv, bv, parts)


# speedup vs baseline: 1.6259x; 1.6259x over previous
import functools

import jax
import jax.numpy as jnp
from jax import lax
from jax.experimental import pallas as pl
from jax.experimental.pallas import tpu as pltpu
from jax.experimental.pallas import tpu_sc as plsc

N = 100000
H = 128
G = 512
R = 8192
NBLK = 13
NPAD = NBLK * R
NC, NS = 2, 16
NW = NC * NS
CH = NPAD // NW
W = 64
NV = CH // 16
NEG = -1e30


def _k1_body(x_ref, wg1_ref, bg1_ref, wg2_ref, bg2_ref, g_ref, gmax_ref, mscr):
    i = pl.program_id(0)
    x = x_ref[...]
    h = jnp.dot(x, wg1_ref[...], preferred_element_type=jnp.float32) + bg1_ref[...]
    h = h * jax.nn.sigmoid(h)
    g = jnp.dot(h, wg2_ref[...], preferred_element_type=jnp.float32) + bg2_ref[...]
    g_ref[...] = g

    @pl.when(i == NBLK - 1)
    def _():
        rows = i * R + lax.broadcasted_iota(jnp.int32, (R, 1), 0)
        g_ref[...] = jnp.where(rows < N, g, NEG)

    @pl.when(i == 0)
    def _():
        mscr[...] = jnp.full((8, 128), NEG, jnp.float32)

    mscr[...] = jnp.maximum(mscr[...], jnp.max(g_ref[...]))

    @pl.when(i == NBLK - 1)
    def _():
        gmax_ref[...] = mscr[...]


def _k3_body(x_ref, e_ref, b_ref, base_ref, wide_ref, wv_ref, bv_ref,
             parts_ref, out_ref):
    i = pl.program_id(0)

    @pl.when(i == 0)
    def _():
        out_ref[...] = jnp.zeros((G, H), jnp.float32)

    x = x_ref[...]
    v = jnp.dot(x, wv_ref[...], preferred_element_type=jnp.float32) + bv_ref[...]
    v = v * jax.nn.sigmoid(v)
    ids = lax.broadcast_in_dim(b_ref[0], (W, R), (0, 1))
    ev = lax.broadcast_in_dim(e_ref[0], (W, R), (0, 1))
    wide = wide_ref[i]
    base = base_ref[i]
    nwin = jnp.where(wide != 0, G // W, 1)
    iota_s = lax.broadcasted_iota(jnp.int32, (W, R), 0)

    def accumulate(vv):
        def body(k, carry):
            wbase = jnp.where(wide != 0, k * W, base)
            ohe = jnp.where(ids == wbase + iota_s, ev, 0.0)
            out_ref[pl.ds(wbase, W), :] += lax.dot_general(
                ohe, vv, (((1,), (0,)), ((), ())),
                preferred_element_type=jnp.float32)
            return carry
        lax.fori_loop(0, nwin, body, 0)

    @pl.when(i < NBLK - 1)
    def _():
        accumulate(v)

    @pl.when(i == NBLK - 1)
    def _():
        rows = i * R + lax.broadcasted_iota(jnp.int32, (R, 1), 0)
        accumulate(jnp.where(rows < N, v, 0.0))
        den = jnp.sum(parts_ref[...], axis=0)
        out_ref[...] = out_ref[...] / (den[:, None] + 1e-16)


def _sc_softmax_body(g_hbm, b_hbm, gmax_hbm, e_hbm, parts_hbm,
                     g_v, b_v, e_v, den_v, gmax_v, sem0, sem1, sem2):
    wid = lax.axis_index("s") * NC + lax.axis_index("c")
    base = wid * CH
    c0 = pltpu.async_copy(g_hbm.at[pl.ds(base, CH)], g_v, sem0)
    c1 = pltpu.async_copy(b_hbm.at[pl.ds(base, CH)], b_v, sem1)
    c2 = pltpu.async_copy(gmax_hbm, gmax_v, sem2)
    z = jnp.zeros((16,), jnp.float32)
    for k in range(G // 16):
        den_v[pl.ds(k * 16, 16)] = z
    c0.wait()
    c1.wait()
    c2.wait()
    gm = gmax_v[...]

    def body(j, carry):
        sl = pl.ds(j * 16, 16)
        ev = jnp.exp(g_v[sl] - gm)
        e_v[sl] = ev
        plsc.addupdate_scatter(den_v, [b_v[sl]], ev)
        return carry

    lax.fori_loop(0, NV, body, 0)
    pltpu.sync_copy(e_v, e_hbm.at[pl.ds(base, CH)])
    pltpu.sync_copy(den_v, parts_hbm.at[wid])


@functools.cache
def _sc_softmax():
    return functools.partial(
        pl.kernel,
        mesh=plsc.VectorSubcoreMesh(core_axis_name="c", subcore_axis_name="s"),
        out_type=[
            jax.ShapeDtypeStruct((NPAD,), jnp.float32),
            jax.ShapeDtypeStruct((NW, G), jnp.float32),
        ],
        scratch_types=[
            pltpu.VMEM((CH,), jnp.float32),
            pltpu.VMEM((CH,), jnp.int32),
            pltpu.VMEM((CH,), jnp.float32),
            pltpu.VMEM((G,), jnp.float32),
            pltpu.VMEM((16,), jnp.float32),
            pltpu.SemaphoreType.DMA,
            pltpu.SemaphoreType.DMA,
            pltpu.SemaphoreType.DMA,
        ],
        compiler_params=pltpu.CompilerParams(needs_layout_passes=False),
    )(_sc_softmax_body)


def _gate_pass(x, Wg1, bg1, Wg2, bg2):
    return pl.pallas_call(
        _k1_body,
        grid=(NBLK,),
        in_specs=[
            pl.BlockSpec((R, H), lambda i: (i, 0)),
            pl.BlockSpec((H, H // 2), lambda i: (0, 0)),
            pl.BlockSpec((1, H // 2), lambda i: (0, 0)),
            pl.BlockSpec((H // 2, 1), lambda i: (0, 0)),
            pl.BlockSpec((1, 1), lambda i: (0, 0)),
        ],
        out_specs=[
            pl.BlockSpec((R, 1), lambda i: (i, 0)),
            pl.BlockSpec((8, 128), lambda i: (0, 0)),
        ],
        out_shape=[
            jax.ShapeDtypeStruct((NPAD, 1), jnp.float32),
            jax.ShapeDtypeStruct((8, 128), jnp.float32),
        ],
        scratch_shapes=[pltpu.VMEM((8, 128), jnp.float32)],
        compiler_params=pltpu.CompilerParams(
            dimension_semantics=("arbitrary",)),
    )(x, Wg1.T, bg1.reshape(1, H // 2), Wg2.T, bg2.reshape(1, 1))


def _pool_pass(x, e, batch2d, base, wide, Wv, bv, parts):
    return pl.pallas_call(
        _k3_body,
        grid=(NBLK,),
        in_specs=[
            pl.BlockSpec((R, H), lambda i: (i, 0)),
            pl.BlockSpec((1, 1, R), lambda i: (i, 0, 0)),
            pl.BlockSpec((1, 1, R), lambda i: (i, 0, 0)),
            pl.BlockSpec((NBLK,), lambda i: (0,), memory_space=pltpu.SMEM),
            pl.BlockSpec((NBLK,), lambda i: (0,), memory_space=pltpu.SMEM),
            pl.BlockSpec((H, H), lambda i: (0, 0)),
            pl.BlockSpec((1, H), lambda i: (0, 0)),
            pl.BlockSpec((NW, G), lambda i: (0, 0)),
        ],
        out_specs=pl.BlockSpec((G, H), lambda i: (0, 0)),
        out_shape=jax.ShapeDtypeStruct((G, H), jnp.float32),
        compiler_params=pltpu.CompilerParams(
            dimension_semantics=("arbitrary",)),
    )(x, e, batch2d, base, wide, Wv.T, bv.reshape(1, H), parts)


def kernel(x, batch, Wg1, bg1, Wg2, bg2, Wv, bv):
    batch_i = batch.astype(jnp.int32)
    batch_p = jnp.pad(batch_i, (0, NPAD - N), mode="edge")
    b2 = batch_p.reshape(NBLK, R)
    lo, hi = b2[:, 0], b2[:, -1]
    base = jnp.minimum(lo & -8, G - W)
    wide = (hi - base >= W).astype(jnp.int32)
    g, gmax = _gate_pass(x, Wg1, bg1, Wg2, bg2)
    e, parts = _sc_softmax()(g.reshape(NPAD), batch_p, gmax.reshape(-1)[:16])
    return _pool_pass(x, e.reshape(NBLK, 1, R), batch_p.reshape(NBLK, 1, R),
                      base, wide, Wv, bv, parts)


# speedup vs baseline: 1.6539x; 1.0172x over previous
import functools

import jax
import jax.numpy as jnp
from jax import lax
from jax.experimental import pallas as pl
from jax.experimental.pallas import tpu as pltpu
from jax.experimental.pallas import tpu_sc as plsc

N = 100000
H = 128
G = 512
R = 8192
NBLK = 13
NPAD = NBLK * R
NC, NS = 2, 16
NW = NC * NS
CH = NPAD // NW
W = 64
NV = CH // 16
NEG = -1e30


def _k1_body(x_ref, wg1_ref, bg1_ref, wg2_ref, bg2_ref, g_ref, gmax_ref, mscr):
    i = pl.program_id(0)
    x = x_ref[...]
    h = jnp.dot(x, wg1_ref[...], preferred_element_type=jnp.float32) + bg1_ref[...]
    h = h * jax.nn.sigmoid(h)
    g = jnp.dot(h, wg2_ref[...], preferred_element_type=jnp.float32) + bg2_ref[...]
    g_ref[...] = g

    @pl.when(i == NBLK - 1)
    def _():
        rows = i * R + lax.broadcasted_iota(jnp.int32, (R, 1), 0)
        g_ref[...] = jnp.where(rows < N, g, NEG)

    @pl.when(i == 0)
    def _():
        mscr[...] = jnp.full((8, 128), NEG, jnp.float32)

    mscr[...] = jnp.maximum(mscr[...], jnp.max(g_ref[...]))

    @pl.when(i == NBLK - 1)
    def _():
        gmax_ref[...] = mscr[...]


def _k3_body(x_ref, e_ref, b_ref, base_ref, wide_ref, wv_ref, bv_ref,
             parts_ref, out_ref):
    i = pl.program_id(0)

    @pl.when(i == 0)
    def _():
        out_ref[...] = jnp.zeros((G, H), jnp.float32)

    x = x_ref[...]
    v = jnp.dot(x, wv_ref[...], preferred_element_type=jnp.float32) + bv_ref[...]
    v = v * jax.nn.sigmoid(v)
    ids = lax.broadcast_in_dim(b_ref[0], (W, R), (0, 1))
    ev = lax.broadcast_in_dim(e_ref[0], (W, R), (0, 1))
    wide = wide_ref[i]
    base = base_ref[i]
    nwin = jnp.where(wide != 0, G // W, 1)
    iota_s = lax.broadcasted_iota(jnp.int32, (W, R), 0)

    def accumulate(vv):
        def body(k, carry):
            wbase = jnp.where(wide != 0, k * W, base)
            ohe = jnp.where(ids == wbase + iota_s, ev, 0.0)
            out_ref[pl.ds(wbase, W), :] += lax.dot_general(
                ohe, vv, (((1,), (0,)), ((), ())),
                preferred_element_type=jnp.float32)
            return carry
        lax.fori_loop(0, nwin, body, 0)

    @pl.when(i < NBLK - 1)
    def _():
        accumulate(v)

    @pl.when(i == NBLK - 1)
    def _():
        rows = i * R + lax.broadcasted_iota(jnp.int32, (R, 1), 0)
        accumulate(jnp.where(rows < N, v, 0.0))
        den = jnp.sum(parts_ref[...], axis=0)
        out_ref[...] = out_ref[...] / (den[:, None] + 1e-16)


def _sc_softmax_body(g_hbm, b_hbm, gmax_hbm, e_hbm, parts_hbm,
                     g_v, b_v, e_v, den_v, gmax_v, sem0, sem1, sem2):
    wid = lax.axis_index("s") * NC + lax.axis_index("c")
    base = wid * CH
    c0 = pltpu.async_copy(g_hbm.at[pl.ds(base, CH)], g_v, sem0)
    c1 = pltpu.async_copy(b_hbm.at[pl.ds(base, CH)], b_v, sem1)
    c2 = pltpu.async_copy(gmax_hbm.at[0, pl.ds(0, 16)], gmax_v, sem2)
    z = jnp.zeros((16,), jnp.float32)
    for k in range(G // 16):
        den_v[pl.ds(k * 16, 16)] = z
    c0.wait()
    c1.wait()
    c2.wait()
    gm = gmax_v[...]

    def body(j, carry):
        sl = pl.ds(j * 16, 16)
        ev = jnp.exp(g_v[sl] - gm)
        e_v[sl] = ev
        plsc.addupdate_scatter(den_v, [b_v[sl]], ev)
        return carry

    lax.fori_loop(0, NV, body, 0)
    pltpu.sync_copy(e_v, e_hbm.at[pl.ds(base, CH)])
    pltpu.sync_copy(den_v, parts_hbm.at[wid])


@functools.cache
def _sc_softmax():
    return functools.partial(
        pl.kernel,
        mesh=plsc.VectorSubcoreMesh(core_axis_name="c", subcore_axis_name="s"),
        out_type=[
            jax.ShapeDtypeStruct((NPAD,), jnp.float32),
            jax.ShapeDtypeStruct((NW, G), jnp.float32),
        ],
        scratch_types=[
            pltpu.VMEM((CH,), jnp.float32),
            pltpu.VMEM((CH,), jnp.int32),
            pltpu.VMEM((CH,), jnp.float32),
            pltpu.VMEM((G,), jnp.float32),
            pltpu.VMEM((16,), jnp.float32),
            pltpu.SemaphoreType.DMA,
            pltpu.SemaphoreType.DMA,
            pltpu.SemaphoreType.DMA,
        ],
        compiler_params=pltpu.CompilerParams(needs_layout_passes=False),
    )(_sc_softmax_body)


def _gate_pass(x, Wg1, bg1, Wg2, bg2):
    return pl.pallas_call(
        _k1_body,
        grid=(NBLK,),
        in_specs=[
            pl.BlockSpec((R, H), lambda i: (i, 0)),
            pl.BlockSpec((H, H // 2), lambda i: (0, 0)),
            pl.BlockSpec((1, H // 2), lambda i: (0, 0)),
            pl.BlockSpec((H // 2, 1), lambda i: (0, 0)),
            pl.BlockSpec((1, 1), lambda i: (0, 0)),
        ],
        out_specs=[
            pl.BlockSpec((R, 1), lambda i: (i, 0)),
            pl.BlockSpec((8, 128), lambda i: (0, 0)),
        ],
        out_shape=[
            jax.ShapeDtypeStruct((NPAD, 1), jnp.float32),
            jax.ShapeDtypeStruct((8, 128), jnp.float32),
        ],
        scratch_shapes=[pltpu.VMEM((8, 128), jnp.float32)],
        compiler_params=pltpu.CompilerParams(
            dimension_semantics=("arbitrary",)),
    )(x, Wg1.T, bg1.reshape(1, H // 2), Wg2.T, bg2.reshape(1, 1))


def _pool_pass(x, e, batch2d, base, wide, Wv, bv, parts):
    return pl.pallas_call(
        _k3_body,
        grid=(NBLK,),
        in_specs=[
            pl.BlockSpec((R, H), lambda i: (i, 0)),
            pl.BlockSpec((1, 1, R), lambda i: (i, 0, 0)),
            pl.BlockSpec((1, 1, R), lambda i: (i, 0, 0)),
            pl.BlockSpec((NBLK,), lambda i: (0,), memory_space=pltpu.SMEM),
            pl.BlockSpec((NBLK,), lambda i: (0,), memory_space=pltpu.SMEM),
            pl.BlockSpec((H, H), lambda i: (0, 0)),
            pl.BlockSpec((1, H), lambda i: (0, 0)),
            pl.BlockSpec((NW, G), lambda i: (0, 0)),
        ],
        out_specs=pl.BlockSpec((G, H), lambda i: (0, 0)),
        out_shape=jax.ShapeDtypeStruct((G, H), jnp.float32),
        compiler_params=pltpu.CompilerParams(
            dimension_semantics=("arbitrary",)),
    )(x, e, batch2d, base, wide, Wv.T, bv.reshape(1, H), parts)


def kernel(x, batch, Wg1, bg1, Wg2, bg2, Wv, bv):
    batch_i = batch.astype(jnp.int32)
    batch_p = jnp.pad(batch_i, (0, NPAD - N), mode="edge")
    b2 = batch_p.reshape(NBLK, R)
    lo, hi = b2[:, 0], b2[:, -1]
    base = jnp.minimum(lo & -8, G - W)
    wide = (hi - base >= W).astype(jnp.int32)
    g, gmax = _gate_pass(x, Wg1, bg1, Wg2, bg2)
    e, parts = _sc_softmax()(g.reshape(NPAD), batch_p, gmax)
    return _pool_pass(x, e.reshape(NBLK, 1, R), batch_p.reshape(NBLK, 1, R),
                      base, wide, Wv, bv, parts)
